# baseline (speedup 1.0000x reference)
import functools

import numpy as np
import jax
import jax.numpy as jnp
from jax import lax
from jax.experimental import pallas as pl
from jax.experimental.pallas import tpu as pltpu
from jax.experimental.pallas import tpu_sc as plsc

_N_LEVELS = 16
_F = 2
_T = 1 << 19
_MASK = _T - 1
_BASE_RES = 16
_PER_LEVEL_SCALE = 1.5
_N = 131072
_P1 = 2654435761 - (1 << 32)
_P2 = 805459861

_NC = 2
_NS = 16
_NW = _NC * _NS
_C = 512
_G = _C // 16
_PW = _N // _NW
_CH = _PW // _C


def _level_consts():
    out = []
    for l in range(_N_LEVELS):
        s = _BASE_RES * (_PER_LEVEL_SCALE ** l) - 1.0
        r = int(np.ceil(s)) + 1
        out.append((float(np.float32(s)), r, r ** 3 > _T))
    return out


_LEVELS = _level_consts()


def _body(xin, tab, out, xv, wv, idxv, subv, rowsv, outv, semA, semB):
    wid = lax.axis_index("s") * _NC + lax.axis_index("c")
    iota = lax.iota(jnp.int32, 16)
    col0 = jnp.zeros((16,), jnp.int32)
    col1 = jnp.full((16,), 1, jnp.int32)
    col2 = jnp.full((16,), 2, jnp.int32)

    def chunk_body(ch, carry):
        base = wid * _PW + ch * _C
        pltpu.sync_copy(xin.at[pl.ds(base, _C)], xv)

        for l, (s, r, hashed) in enumerate(_LEVELS):

            def idx_body(g, c, s=s, r=r, hashed=hashed, l=l):
                o = g * 16
                rows = o + iota
                px = plsc.load_gather(xv, [rows, col0])
                py = plsc.load_gather(xv, [rows, col1])
                pz = plsc.load_gather(xv, [rows, col2])
                px = jnp.minimum(jnp.maximum(px, 0.0), 1.0)
                py = jnp.minimum(jnp.maximum(py, 0.0), 1.0)
                pz = jnp.minimum(jnp.maximum(pz, 0.0), 1.0)
                fx = px * s + 0.5
                fy = py * s + 0.5
                fz = pz * s + 0.5
                ix = fx.astype(jnp.int32)
                iy = fy.astype(jnp.int32)
                iz = fz.astype(jnp.int32)
                wv[0, pl.ds(o, 16)] = fx - ix.astype(jnp.float32)
                wv[1, pl.ds(o, 16)] = fy - iy.astype(jnp.float32)
                wv[2, pl.ds(o, 16)] = fz - iz.astype(jnp.float32)
                qx = ix + 1
                qy = iy + 1
                qz = iz + 1
                if hashed:
                    a = (ix, qx)
                    b = (iy * _P1, qy * _P1)
                    cc = (iz * _P2, qz * _P2)
                    for k in range(8):
                        h = a[k & 1] ^ b[(k >> 1) & 1] ^ cc[(k >> 2) & 1]
                        idx = (h & _MASK) + l * _T
                        idxv[g, pl.ds(k * 16, 16)] = lax.shift_right_logical(idx, 3)
                        subv[g, pl.ds(k * 16, 16)] = (idx & 7) * 2
                else:
                    a = (ix, qx)
                    b = (iy * r, qy * r)
                    cc = (iz * (r * r) + l * _T, qz * (r * r) + l * _T)
                    for k in range(8):
                        idx = a[k & 1] + b[(k >> 1) & 1] + cc[(k >> 2) & 1]
                        idxv[g, pl.ds(k * 16, 16)] = lax.shift_right_logical(idx, 3)
                        subv[g, pl.ds(k * 16, 16)] = (idx & 7) * 2
                return c

            lax.fori_loop(0, _G, idx_body, 0)

            def accum_group(g, l=l):
                o = g * 16
                wx = wv[0, pl.ds(o, 16)]
                wy = wv[1, pl.ds(o, 16)]
                wz = wv[2, pl.ds(o, 16)]
                ux = 1.0 - wx
                uy = 1.0 - wy
                uz = 1.0 - wz
                gfull = jnp.full((16,), 0, jnp.int32) + g
                acc0 = jnp.zeros((16,), jnp.float32)
                acc1 = jnp.zeros((16,), jnp.float32)
                for k in range(8):
                    wc = ((wx if (k & 1) else ux)
                          * (wy if ((k >> 1) & 1) else uy)
                          * (wz if ((k >> 2) & 1) else uz))
                    rrows = k * 16 + iota
                    sub = subv[g, pl.ds(k * 16, 16)]
                    f0 = plsc.load_gather(rowsv, [gfull, rrows, sub])
                    f1 = plsc.load_gather(rowsv, [gfull, rrows, sub + 1])
                    acc0 = acc0 + f0 * wc
                    acc1 = acc1 + f1 * wc
                prow = o + iota
                plsc.store_scatter(outv, [prow, jnp.full((16,), 2 * l, jnp.int32)], acc0)
                plsc.store_scatter(outv, [prow, jnp.full((16,), 2 * l + 1, jnp.int32)], acc1)

            pltpu.async_copy(tab.at[idxv.at[0]], rowsv.at[0], semA)
            pltpu.async_copy(tab.at[idxv.at[1]], rowsv.at[1], semB)

            def acc_pair(t, c, l=l):
                g0 = 2 * t
                g1 = g0 + 1
                pltpu.make_async_copy(tab.at[idxv.at[g0]], rowsv.at[g0], semA).wait()

                @pl.when(t < _G // 2 - 1)
                def _():
                    pltpu.async_copy(tab.at[idxv.at[g0 + 2]], rowsv.at[g0 + 2], semA)

                accum_group(g0, l)
                pltpu.make_async_copy(tab.at[idxv.at[g1]], rowsv.at[g1], semB).wait()

                @pl.when(t < _G // 2 - 1)
                def _():
                    pltpu.async_copy(tab.at[idxv.at[g1 + 2]], rowsv.at[g1 + 2], semB)

                accum_group(g1, l)
                return c

            lax.fori_loop(0, _G // 2, acc_pair, 0)

        pltpu.sync_copy(outv, out.at[pl.ds(base, _C)])
        return carry

    lax.fori_loop(0, _CH, chunk_body, 0)


_sc_encode = pl.kernel(
    _body,
    out_type=jax.ShapeDtypeStruct((_N, 2 * _N_LEVELS), jnp.float32),
    mesh=plsc.VectorSubcoreMesh(core_axis_name="c", subcore_axis_name="s"),
    compiler_params=pltpu.CompilerParams(
        needs_layout_passes=False, use_tc_tiling_on_sc=False),
    scratch_types=[
        pltpu.VMEM((_C, 3), jnp.float32),
        pltpu.VMEM((3, _C), jnp.float32),
        pltpu.VMEM((_G, 128), jnp.int32),
        pltpu.VMEM((_G, 128), jnp.int32),
        pltpu.VMEM((_G, 128, 16), jnp.float32),
        pltpu.VMEM((_C, 2 * _N_LEVELS), jnp.float32),
        pltpu.SemaphoreType.DMA,
        pltpu.SemaphoreType.DMA,
    ],
)


def kernel(x, table):
    tab = table.reshape(_N_LEVELS * _T * _F // 16, 16)
    return _sc_encode(x, tab)


# speedup vs baseline: 1.0585x; 1.0585x over previous
import functools

import numpy as np
import jax
import jax.numpy as jnp
from jax import lax
from jax.experimental import pallas as pl
from jax.experimental.pallas import tpu as pltpu
from jax.experimental.pallas import tpu_sc as plsc

_N_LEVELS = 16
_F = 2
_T = 1 << 19
_MASK = _T - 1
_BASE_RES = 16
_PER_LEVEL_SCALE = 1.5
_N = 131072
_P1 = 2654435761 - (1 << 32)
_P2 = 805459861

_NC = 2
_NS = 16
_NW = _NC * _NS
_C = 512
_G = _C // 16
_PW = _N // _NW
_CH = _PW // _C


def _level_consts():
    out = []
    for l in range(_N_LEVELS):
        s = _BASE_RES * (_PER_LEVEL_SCALE ** l) - 1.0
        r = int(np.ceil(s)) + 1
        out.append((float(np.float32(s)), r, r ** 3 > _T))
    return out


_LEVELS = _level_consts()


def _body(xin, tab, out, xv, wv, idxv, subv, rowsv, outv, semA, semB):
    wid = lax.axis_index("s") * _NC + lax.axis_index("c")
    iota = lax.iota(jnp.int32, 16)
    col0 = jnp.zeros((16,), jnp.int32)
    col1 = jnp.full((16,), 1, jnp.int32)
    col2 = jnp.full((16,), 2, jnp.int32)

    def chunk_body(ch, carry):
        base = wid * _PW + ch * _C
        pltpu.sync_copy(xin.at[pl.ds(base, _C)], xv)

        for l, (s, r, hashed) in enumerate(_LEVELS):

            def idx_body(g, c, s=s, r=r, hashed=hashed, l=l):
                o = g * 16
                rows = o + iota
                px = plsc.load_gather(xv, [rows, col0])
                py = plsc.load_gather(xv, [rows, col1])
                pz = plsc.load_gather(xv, [rows, col2])
                px = jnp.minimum(jnp.maximum(px, 0.0), 1.0)
                py = jnp.minimum(jnp.maximum(py, 0.0), 1.0)
                pz = jnp.minimum(jnp.maximum(pz, 0.0), 1.0)
                fx = px * s + 0.5
                fy = py * s + 0.5
                fz = pz * s + 0.5
                ix = fx.astype(jnp.int32)
                iy = fy.astype(jnp.int32)
                iz = fz.astype(jnp.int32)
                wv[0, pl.ds(o, 16)] = fx - ix.astype(jnp.float32)
                wv[1, pl.ds(o, 16)] = fy - iy.astype(jnp.float32)
                wv[2, pl.ds(o, 16)] = fz - iz.astype(jnp.float32)
                qx = ix + 1
                qy = iy + 1
                qz = iz + 1
                if hashed:
                    a = (ix, qx)
                    b = (iy * _P1, qy * _P1)
                    cc = (iz * _P2, qz * _P2)
                    for k in range(8):
                        h = a[k & 1] ^ b[(k >> 1) & 1] ^ cc[(k >> 2) & 1]
                        idx = (h & _MASK) + l * _T
                        idxv[pl.ds(g * 128 + k * 16, 16)] = lax.shift_right_logical(idx, 3)
                        subv[pl.ds(g * 128 + k * 16, 16)] = (idx & 7) * 2
                else:
                    a = (ix, qx)
                    b = (iy * r, qy * r)
                    cc = (iz * (r * r) + l * _T, qz * (r * r) + l * _T)
                    for k in range(8):
                        idx = a[k & 1] + b[(k >> 1) & 1] + cc[(k >> 2) & 1]
                        idxv[pl.ds(g * 128 + k * 16, 16)] = lax.shift_right_logical(idx, 3)
                        subv[pl.ds(g * 128 + k * 16, 16)] = (idx & 7) * 2
                return c

            lax.fori_loop(0, _G, idx_body, 0)

            pltpu.async_copy(tab.at[idxv], rowsv, semA).wait()

            def acc_body(g, c, l=l):
                o = g * 16
                wx = wv[0, pl.ds(o, 16)]
                wy = wv[1, pl.ds(o, 16)]
                wz = wv[2, pl.ds(o, 16)]
                ux = 1.0 - wx
                uy = 1.0 - wy
                uz = 1.0 - wz
                acc0 = jnp.zeros((16,), jnp.float32)
                acc1 = jnp.zeros((16,), jnp.float32)
                for k in range(8):
                    wc = ((wx if (k & 1) else ux)
                          * (wy if ((k >> 1) & 1) else uy)
                          * (wz if ((k >> 2) & 1) else uz))
                    rrows = g * 128 + k * 16 + iota
                    sub = subv[pl.ds(g * 128 + k * 16, 16)]
                    f0 = plsc.load_gather(rowsv, [rrows, sub])
                    f1 = plsc.load_gather(rowsv, [rrows, sub + 1])
                    acc0 = acc0 + f0 * wc
                    acc1 = acc1 + f1 * wc
                prow = o + iota
                plsc.store_scatter(outv, [prow, jnp.full((16,), 2 * l, jnp.int32)], acc0)
                plsc.store_scatter(outv, [prow, jnp.full((16,), 2 * l + 1, jnp.int32)], acc1)
                return c

            lax.fori_loop(0, _G, acc_body, 0)

        pltpu.sync_copy(outv, out.at[pl.ds(base, _C)])
        return carry

    lax.fori_loop(0, _CH, chunk_body, 0)


_sc_encode = pl.kernel(
    _body,
    out_type=jax.ShapeDtypeStruct((_N, 2 * _N_LEVELS), jnp.float32),
    mesh=plsc.VectorSubcoreMesh(core_axis_name="c", subcore_axis_name="s"),
    compiler_params=pltpu.CompilerParams(
        needs_layout_passes=False, use_tc_tiling_on_sc=False),
    scratch_types=[
        pltpu.VMEM((_C, 3), jnp.float32),
        pltpu.VMEM((3, _C), jnp.float32),
        pltpu.VMEM((8 * _C,), jnp.int32),
        pltpu.VMEM((8 * _C,), jnp.int32),
        pltpu.VMEM((8 * _C, 16), jnp.float32),
        pltpu.VMEM((_C, 2 * _N_LEVELS), jnp.float32),
        pltpu.SemaphoreType.DMA,
        pltpu.SemaphoreType.DMA,
    ],
)


def kernel(x, table):
    tab = table.reshape(_N_LEVELS * _T * _F // 16, 16)
    return _sc_encode(x, tab)


# speedup vs baseline: 6.2922x; 5.9442x over previous
import functools

import numpy as np
import jax
import jax.numpy as jnp
from jax import lax
from jax.experimental import pallas as pl
from jax.experimental.pallas import tpu as pltpu
from jax.experimental.pallas import tpu_sc as plsc

_N_LEVELS = 16
_F = 2
_T = 1 << 19
_MASK = _T - 1
_BASE_RES = 16
_PER_LEVEL_SCALE = 1.5
_N = 131072
_P1 = 2654435761 - (1 << 32)
_P2 = 805459861

_NC = 2
_NS = 16
_NW = _NC * _NS
_C = 256
_G = _C // 16
_PW = _N // _NW
_CH = _PW // _C
_LVL_ROWS = _T * _F // 16
_F1_OFF = _T // 16


def _level_consts():
    out = []
    for l in range(_N_LEVELS):
        s = _BASE_RES * (_PER_LEVEL_SCALE ** l) - 1.0
        r = int(np.ceil(s)) + 1
        out.append((float(np.float32(s)), r, r ** 3 > _T))
    return out


_LEVELS = _level_consts()


def _body(xin, tab, out, xv, wv, idxv, subv, rowsv, outv, semA, semB):
    wid = lax.axis_index("s") * _NC + lax.axis_index("c")
    iota = lax.iota(jnp.int32, 16)
    col0 = jnp.zeros((16,), jnp.int32)
    col1 = jnp.full((16,), 1, jnp.int32)
    col2 = jnp.full((16,), 2, jnp.int32)

    def chunk_body(ch, carry):
        base = wid * _PW + ch * _C
        pltpu.sync_copy(xin.at[pl.ds(base, _C)], xv)

        for l, (s, r, hashed) in enumerate(_LEVELS):

            def idx_body(g, c, s=s, r=r, hashed=hashed, l=l):
                o = g * 16
                rows = o + iota
                px = plsc.load_gather(xv, [rows, col0])
                py = plsc.load_gather(xv, [rows, col1])
                pz = plsc.load_gather(xv, [rows, col2])
                px = jnp.minimum(jnp.maximum(px, 0.0), 1.0)
                py = jnp.minimum(jnp.maximum(py, 0.0), 1.0)
                pz = jnp.minimum(jnp.maximum(pz, 0.0), 1.0)
                fx = px * s + 0.5
                fy = py * s + 0.5
                fz = pz * s + 0.5
                ix = fx.astype(jnp.int32)
                iy = fy.astype(jnp.int32)
                iz = fz.astype(jnp.int32)
                wv[0, pl.ds(o, 16)] = fx - ix.astype(jnp.float32)
                wv[1, pl.ds(o, 16)] = fy - iy.astype(jnp.float32)
                wv[2, pl.ds(o, 16)] = fz - iz.astype(jnp.float32)
                qx = ix + 1
                qy = iy + 1
                qz = iz + 1
                if hashed:
                    a = (ix, qx)
                    b = (iy * _P1, qy * _P1)
                    cc = (iz * _P2, qz * _P2)
                else:
                    a = (ix, qx)
                    b = (iy * r, qy * r)
                    cc = (iz * (r * r), qz * (r * r))
                for k in range(8):
                    if hashed:
                        p = (a[k & 1] ^ b[(k >> 1) & 1] ^ cc[(k >> 2) & 1]) & _MASK
                    else:
                        p = a[k & 1] + b[(k >> 1) & 1] + cc[(k >> 2) & 1]
                    row0 = ((l * _LVL_ROWS)
                            + lax.shift_right_logical(p, 7) * 16
                            + (lax.shift_right_logical(p, 4) & 7))
                    q = g * 256 + k * 32
                    idxv[pl.ds(q, 16)] = row0
                    idxv[pl.ds(q + 16, 16)] = row0 + 8
                    subv[pl.ds(g * 128 + k * 16, 16)] = p & 15
                return c

            lax.fori_loop(0, _G, idx_body, 0)

            pltpu.async_copy(tab.at[idxv], rowsv, semA).wait()

            def acc_body(g, c, l=l):
                o = g * 16
                wx = wv[0, pl.ds(o, 16)]
                wy = wv[1, pl.ds(o, 16)]
                wz = wv[2, pl.ds(o, 16)]
                ux = 1.0 - wx
                uy = 1.0 - wy
                uz = 1.0 - wz
                acc0 = jnp.zeros((16,), jnp.float32)
                acc1 = jnp.zeros((16,), jnp.float32)
                for k in range(8):
                    wc = ((wx if (k & 1) else ux)
                          * (wy if ((k >> 1) & 1) else uy)
                          * (wz if ((k >> 2) & 1) else uz))
                    q = g * 256 + k * 32
                    sub = subv[pl.ds(g * 128 + k * 16, 16)]
                    f0 = plsc.load_gather(rowsv, [q + iota, sub])
                    f1 = plsc.load_gather(rowsv, [q + 16 + iota, sub])
                    acc0 = acc0 + f0 * wc
                    acc1 = acc1 + f1 * wc
                prow = o + iota
                plsc.store_scatter(outv, [prow, jnp.full((16,), 2 * l, jnp.int32)], acc0)
                plsc.store_scatter(outv, [prow, jnp.full((16,), 2 * l + 1, jnp.int32)], acc1)
                return c

            lax.fori_loop(0, _G, acc_body, 0)

        pltpu.sync_copy(outv, out.at[pl.ds(base, _C)])
        return carry

    lax.fori_loop(0, _CH, chunk_body, 0)


_sc_encode = pl.kernel(
    _body,
    out_type=jax.ShapeDtypeStruct((_N, 2 * _N_LEVELS), jnp.float32),
    mesh=plsc.VectorSubcoreMesh(core_axis_name="c", subcore_axis_name="s"),
    compiler_params=pltpu.CompilerParams(
        needs_layout_passes=False, use_tc_tiling_on_sc=False),
    scratch_types=[
        pltpu.VMEM((_C, 3), jnp.float32),
        pltpu.VMEM((3, _C), jnp.float32),
        pltpu.VMEM((16 * _C,), jnp.int32),
        pltpu.VMEM((8 * _C,), jnp.int32),
        pltpu.VMEM((16 * _C, 16), jnp.float32),
        pltpu.VMEM((_C, 2 * _N_LEVELS), jnp.float32),
        pltpu.SemaphoreType.DMA,
        pltpu.SemaphoreType.DMA,
    ],
)


def kernel(x, table):
    tab = (table.reshape(_N_LEVELS, _T // 128, 128, _F)
           .transpose(0, 1, 3, 2)
           .reshape(_N_LEVELS * _F * _T // 16, 16))
    return _sc_encode(x, tab)


# speedup vs baseline: 8.1267x; 1.2916x over previous
import functools

import numpy as np
import jax
import jax.numpy as jnp
from jax import lax
from jax.experimental import pallas as pl
from jax.experimental.pallas import tpu as pltpu
from jax.experimental.pallas import tpu_sc as plsc

_N_LEVELS = 16
_F = 2
_T = 1 << 19
_MASK = _T - 1
_BASE_RES = 16
_PER_LEVEL_SCALE = 1.5
_N = 131072
_P1 = 2654435761 - (1 << 32)
_P2 = 805459861

_NC = 2
_NS = 16
_NW = _NC * _NS
_C = 128
_G = _C // 16
_PW = _N // _NW
_CH = _PW // _C
_LVL_ROWS = _T * _F // 16


def _level_consts():
    out = []
    for l in range(_N_LEVELS):
        s = _BASE_RES * (_PER_LEVEL_SCALE ** l) - 1.0
        r = int(np.ceil(s)) + 1
        out.append((float(np.float32(s)), r, r ** 3 > _T))
    return out


_LEVELS = _level_consts()


def _body(xin, tab, out, xv, wvA, wvB, idxA, idxB, subA, subB, rowsA, rowsB,
          outv, semA, semB):
    wid = lax.axis_index("s") * _NC + lax.axis_index("c")
    iota = lax.iota(jnp.int32, 16)
    col0 = jnp.zeros((16,), jnp.int32)
    col1 = jnp.full((16,), 1, jnp.int32)
    col2 = jnp.full((16,), 2, jnp.int32)

    def idx_level(l, idxv, subv, wv):
        s, r, hashed = _LEVELS[l]

        def idx_body(g, c):
            o = g * 16
            rows = o + iota
            px = plsc.load_gather(xv, [rows, col0])
            py = plsc.load_gather(xv, [rows, col1])
            pz = plsc.load_gather(xv, [rows, col2])
            px = jnp.minimum(jnp.maximum(px, 0.0), 1.0)
            py = jnp.minimum(jnp.maximum(py, 0.0), 1.0)
            pz = jnp.minimum(jnp.maximum(pz, 0.0), 1.0)
            fx = px * s + 0.5
            fy = py * s + 0.5
            fz = pz * s + 0.5
            ix = fx.astype(jnp.int32)
            iy = fy.astype(jnp.int32)
            iz = fz.astype(jnp.int32)
            wv[0, pl.ds(o, 16)] = fx - ix.astype(jnp.float32)
            wv[1, pl.ds(o, 16)] = fy - iy.astype(jnp.float32)
            wv[2, pl.ds(o, 16)] = fz - iz.astype(jnp.float32)
            qx = ix + 1
            qy = iy + 1
            qz = iz + 1
            if hashed:
                a = (ix, qx)
                b = (iy * _P1, qy * _P1)
                cc = (iz * _P2, qz * _P2)
            else:
                a = (ix, qx)
                b = (iy * r, qy * r)
                cc = (iz * (r * r), qz * (r * r))
            for k in range(8):
                if hashed:
                    p = (a[k & 1] ^ b[(k >> 1) & 1] ^ cc[(k >> 2) & 1]) & _MASK
                else:
                    p = a[k & 1] + b[(k >> 1) & 1] + cc[(k >> 2) & 1]
                t = lax.shift_right_logical(p, 4)
                row0 = (t + (t & ~7)) + (l * _LVL_ROWS)
                q = g * 256 + k * 32
                idxv[pl.ds(q, 16)] = row0
                idxv[pl.ds(q + 16, 16)] = row0 + 8
                subv[pl.ds(g * 128 + k * 16, 16)] = p & 15
            return c

        lax.fori_loop(0, _G, idx_body, 0)

    def acc_level(l, subv, rowsv, wv):
        def acc_body(g, c):
            o = g * 16
            wx = wv[0, pl.ds(o, 16)]
            wy = wv[1, pl.ds(o, 16)]
            wz = wv[2, pl.ds(o, 16)]
            ux = 1.0 - wx
            uy = 1.0 - wy
            uz = 1.0 - wz
            acc0 = jnp.zeros((16,), jnp.float32)
            acc1 = jnp.zeros((16,), jnp.float32)
            for k in range(8):
                wc = ((wx if (k & 1) else ux)
                      * (wy if ((k >> 1) & 1) else uy)
                      * (wz if ((k >> 2) & 1) else uz))
                q = g * 256 + k * 32
                sub = subv[pl.ds(g * 128 + k * 16, 16)]
                f0 = plsc.load_gather(rowsv, [q + iota, sub])
                f1 = plsc.load_gather(rowsv, [q + 16 + iota, sub])
                acc0 = acc0 + f0 * wc
                acc1 = acc1 + f1 * wc
            prow = o + iota
            plsc.store_scatter(outv, [prow, jnp.full((16,), 2 * l, jnp.int32)], acc0)
            plsc.store_scatter(outv, [prow, jnp.full((16,), 2 * l + 1, jnp.int32)], acc1)
            return c

        lax.fori_loop(0, _G, acc_body, 0)

    bufs = ((idxA, subA, rowsA, semA, wvA), (idxB, subB, rowsB, semB, wvB))

    def chunk_body(ch, carry):
        base = wid * _PW + ch * _C
        pltpu.sync_copy(xin.at[pl.ds(base, _C)], xv)

        idx_level(0, idxA, subA, wvA)
        pltpu.async_copy(tab.at[idxA], rowsA, semA)

        for l in range(_N_LEVELS):
            idxc, subc, rowsc, semc, wvc = bufs[l % 2]
            idxn, subn, rowsn, semn, wvn = bufs[(l + 1) % 2]
            if l + 1 < _N_LEVELS:
                idx_level(l + 1, idxn, subn, wvn)
                pltpu.async_copy(tab.at[idxn], rowsn, semn)
            pltpu.make_async_copy(tab.at[idxc], rowsc, semc).wait()
            acc_level(l, subc, rowsc, wvc)

        pltpu.sync_copy(outv, out.at[pl.ds(base, _C)])
        return carry

    lax.fori_loop(0, _CH, chunk_body, 0)


_sc_encode = pl.kernel(
    _body,
    out_type=jax.ShapeDtypeStruct((_N, 2 * _N_LEVELS), jnp.float32),
    mesh=plsc.VectorSubcoreMesh(core_axis_name="c", subcore_axis_name="s"),
    compiler_params=pltpu.CompilerParams(
        needs_layout_passes=False, use_tc_tiling_on_sc=False),
    scratch_types=[
        pltpu.VMEM((_C, 3), jnp.float32),
        pltpu.VMEM((3, _C), jnp.float32),
        pltpu.VMEM((3, _C), jnp.float32),
        pltpu.VMEM((16 * _C,), jnp.int32),
        pltpu.VMEM((16 * _C,), jnp.int32),
        pltpu.VMEM((8 * _C,), jnp.int32),
        pltpu.VMEM((8 * _C,), jnp.int32),
        pltpu.VMEM((16 * _C, 16), jnp.float32),
        pltpu.VMEM((16 * _C, 16), jnp.float32),
        pltpu.VMEM((_C, 2 * _N_LEVELS), jnp.float32),
        pltpu.SemaphoreType.DMA,
        pltpu.SemaphoreType.DMA,
    ],
)


def kernel(x, table):
    tab = (table.reshape(_N_LEVELS, _T // 128, 128, _F)
           .transpose(0, 1, 3, 2)
           .reshape(_N_LEVELS * _F * _T // 16, 16))
    return _sc_encode(x, tab)


# speedup vs baseline: 11.8158x; 1.4539x over previous
import functools

import numpy as np
import jax
import jax.numpy as jnp
from jax import lax
from jax.experimental import pallas as pl
from jax.experimental.pallas import tpu as pltpu
from jax.experimental.pallas import tpu_sc as plsc

_N_LEVELS = 16
_F = 2
_T = 1 << 19
_MASK = _T - 1
_BASE_RES = 16
_PER_LEVEL_SCALE = 1.5
_N = 131072
_P1 = 2654435761 - (1 << 32)
_P2 = 805459861

_NC = 2
_NS = 16
_NW = _NC * _NS
_C = 128
_G = _C // 16
_PW = _N // _NW
_CH = _PW // _C
_LVL_ROWS = _T * _F // 16

_WORDS = _N_LEVELS * _F * _T
_WPS = _WORDS // _NS
_SPAN = 4096
_STEPS = _WPS // _SPAN


def _level_consts():
    out = []
    for l in range(_N_LEVELS):
        s = _BASE_RES * (_PER_LEVEL_SCALE ** l) - 1.0
        r = int(np.ceil(s)) + 1
        out.append((float(np.float32(s)), r, r ** 3 > _T))
    return out


_LEVELS = _level_consts()


def _interleave_body(tin, tout, bufA, bufB, ovA, ovB, semA, semB):
    sid = lax.axis_index("s")
    cid = lax.axis_index("c")
    iota = lax.iota(jnp.int32, 16)
    idx2 = iota * 2
    idx2p1 = idx2 + 1
    wbase = sid * _WPS
    obase = cid * _WORDS + sid * _WPS

    def start_in(st, buf, sem):
        pltpu.async_copy(tin.at[pl.ds(wbase + st * _SPAN, _SPAN)], buf, sem)

    def wait_in(st, buf, sem):
        pltpu.make_async_copy(
            tin.at[pl.ds(wbase + st * _SPAN, _SPAN)], buf, sem).wait()

    def process(buf, ov):
        def unit(u, c):
            boff = (u // 8) * 256
            j16 = (u % 8) * 16
            f0 = buf[pl.ds(boff + j16, 16)]
            f1 = buf[pl.ds(boff + 128 + j16, 16)]
            o = boff + 2 * j16
            plsc.store_scatter(ov, [idx2 + o], f0)
            plsc.store_scatter(ov, [idx2p1 + o], f1)
            return c

        lax.fori_loop(0, _SPAN // 32, unit, 0)

    start_in(0, bufA, semA)

    def step_body(t, c):
        st = 2 * t

        @pl.when(st + 1 < _STEPS)
        def _():
            start_in(st + 1, bufB, semB)

        wait_in(st, bufA, semA)
        process(bufA, ovA)
        pltpu.sync_copy(ovA, tout.at[pl.ds(obase + st * _SPAN, _SPAN)])

        @pl.when(st + 2 < _STEPS)
        def _():
            start_in(st + 2, bufA, semA)

        @pl.when(st + 1 < _STEPS)
        def _():
            wait_in(st + 1, bufB, semB)
            process(bufB, ovB)
            pltpu.sync_copy(ovB, tout.at[pl.ds(obase + (st + 1) * _SPAN, _SPAN)])

        return c

    lax.fori_loop(0, (_STEPS + 1) // 2, step_body, 0)


def _body(xin, tab, out, xv, wvA, wvB, idxA, idxB, subA, subB, rowsA, rowsB,
          outv, semA, semB):
    wid = lax.axis_index("s") * _NC + lax.axis_index("c")
    crows = lax.axis_index("c") * (_WORDS // 16)
    iota = lax.iota(jnp.int32, 16)
    col0 = jnp.zeros((16,), jnp.int32)
    col1 = jnp.full((16,), 1, jnp.int32)
    col2 = jnp.full((16,), 2, jnp.int32)

    def idx_level(l, idxv, subv, wv):
        s, r, hashed = _LEVELS[l]

        def idx_body(g, c):
            o = g * 16
            rows = o + iota
            px = plsc.load_gather(xv, [rows, col0])
            py = plsc.load_gather(xv, [rows, col1])
            pz = plsc.load_gather(xv, [rows, col2])
            px = jnp.minimum(jnp.maximum(px, 0.0), 1.0)
            py = jnp.minimum(jnp.maximum(py, 0.0), 1.0)
            pz = jnp.minimum(jnp.maximum(pz, 0.0), 1.0)
            fx = px * s + 0.5
            fy = py * s + 0.5
            fz = pz * s + 0.5
            ix = fx.astype(jnp.int32)
            iy = fy.astype(jnp.int32)
            iz = fz.astype(jnp.int32)
            wv[0, pl.ds(o, 16)] = fx - ix.astype(jnp.float32)
            wv[1, pl.ds(o, 16)] = fy - iy.astype(jnp.float32)
            wv[2, pl.ds(o, 16)] = fz - iz.astype(jnp.float32)
            qx = ix + 1
            qy = iy + 1
            qz = iz + 1
            if hashed:
                a = (ix, qx)
                b = (iy * _P1, qy * _P1)
                cc = (iz * _P2, qz * _P2)
            else:
                a = (ix, qx)
                b = (iy * r, qy * r)
                cc = (iz * (r * r), qz * (r * r))
            for k in range(8):
                if hashed:
                    p = (a[k & 1] ^ b[(k >> 1) & 1] ^ cc[(k >> 2) & 1]) & _MASK
                else:
                    p = a[k & 1] + b[(k >> 1) & 1] + cc[(k >> 2) & 1]
                row0 = lax.shift_right_logical(p, 3) + (crows + l * _LVL_ROWS)
                q = g * 128 + k * 16
                idxv[pl.ds(q, 16)] = row0
                subv[pl.ds(q, 16)] = (p & 7) * 2
            return c

        lax.fori_loop(0, _G, idx_body, 0)

    def acc_level(l, subv, rowsv, wv):
        def acc_body(g, c):
            o = g * 16
            wx = wv[0, pl.ds(o, 16)]
            wy = wv[1, pl.ds(o, 16)]
            wz = wv[2, pl.ds(o, 16)]
            ux = 1.0 - wx
            uy = 1.0 - wy
            uz = 1.0 - wz
            acc0 = jnp.zeros((16,), jnp.float32)
            acc1 = jnp.zeros((16,), jnp.float32)
            for k in range(8):
                wc = ((wx if (k & 1) else ux)
                      * (wy if ((k >> 1) & 1) else uy)
                      * (wz if ((k >> 2) & 1) else uz))
                q = g * 128 + k * 16
                sub = subv[pl.ds(q, 16)]
                f0 = plsc.load_gather(rowsv, [q + iota, sub])
                f1 = plsc.load_gather(rowsv, [q + iota, sub + 1])
                acc0 = acc0 + f0 * wc
                acc1 = acc1 + f1 * wc
            prow = o + iota
            plsc.store_scatter(outv, [prow, jnp.full((16,), 2 * l, jnp.int32)], acc0)
            plsc.store_scatter(outv, [prow, jnp.full((16,), 2 * l + 1, jnp.int32)], acc1)
            return c

        lax.fori_loop(0, _G, acc_body, 0)

    bufs = ((idxA, subA, rowsA, semA, wvA), (idxB, subB, rowsB, semB, wvB))

    def chunk_body(ch, carry):
        base = wid * _PW + ch * _C
        pltpu.sync_copy(xin.at[pl.ds(base, _C)], xv)

        idx_level(0, idxA, subA, wvA)
        pltpu.async_copy(tab.at[idxA], rowsA, semA)

        for l in range(_N_LEVELS):
            idxc, subc, rowsc, semc, wvc = bufs[l % 2]
            idxn, subn, rowsn, semn, wvn = bufs[(l + 1) % 2]
            if l + 1 < _N_LEVELS:
                idx_level(l + 1, idxn, subn, wvn)
                pltpu.async_copy(tab.at[idxn], rowsn, semn)
            pltpu.make_async_copy(tab.at[idxc], rowsc, semc).wait()
            acc_level(l, subc, rowsc, wvc)

        pltpu.sync_copy(outv, out.at[pl.ds(base, _C)])
        return carry

    lax.fori_loop(0, _CH, chunk_body, 0)


_sc_interleave = pl.kernel(
    _interleave_body,
    out_type=jax.ShapeDtypeStruct((_NC * _WORDS,), jnp.float32),
    mesh=plsc.VectorSubcoreMesh(core_axis_name="c", subcore_axis_name="s"),
    compiler_params=pltpu.CompilerParams(
        needs_layout_passes=False, use_tc_tiling_on_sc=False),
    scratch_types=[
        pltpu.VMEM((_SPAN,), jnp.float32),
        pltpu.VMEM((_SPAN,), jnp.float32),
        pltpu.VMEM((_SPAN,), jnp.float32),
        pltpu.VMEM((_SPAN,), jnp.float32),
        pltpu.SemaphoreType.DMA,
        pltpu.SemaphoreType.DMA,
    ],
)

_sc_encode = pl.kernel(
    _body,
    out_type=jax.ShapeDtypeStruct((_N, 2 * _N_LEVELS), jnp.float32),
    mesh=plsc.VectorSubcoreMesh(core_axis_name="c", subcore_axis_name="s"),
    compiler_params=pltpu.CompilerParams(
        needs_layout_passes=False, use_tc_tiling_on_sc=False),
    scratch_types=[
        pltpu.VMEM((_C, 3), jnp.float32),
        pltpu.VMEM((3, _C), jnp.float32),
        pltpu.VMEM((3, _C), jnp.float32),
        pltpu.VMEM((8 * _C,), jnp.int32),
        pltpu.VMEM((8 * _C,), jnp.int32),
        pltpu.VMEM((8 * _C,), jnp.int32),
        pltpu.VMEM((8 * _C,), jnp.int32),
        pltpu.VMEM((8 * _C, 16), jnp.float32),
        pltpu.VMEM((8 * _C, 16), jnp.float32),
        pltpu.VMEM((_C, 2 * _N_LEVELS), jnp.float32),
        pltpu.SemaphoreType.DMA,
        pltpu.SemaphoreType.DMA,
    ],
)


def kernel(x, table):
    tabn = (table.reshape(_N_LEVELS, _T // 128, 128, _F)
            .transpose(0, 1, 3, 2)
            .reshape(_WORDS))
    tabi = _sc_interleave(tabn).reshape(_NC * _WORDS // 16, 16)
    return _sc_encode(x, tabi)
